# baseline (speedup 1.0000x reference)
import functools

import jax
import jax.numpy as jnp
from jax import lax
from jax.experimental import pallas as pl
from jax.experimental.pallas import tpu as pltpu
from jax.experimental.pallas import tpu_sc as plsc

N = 10000
E = 320000
D = 128
NC = 2
NS = 16
NW = NC * NS
EW = E // NW
CHUNK = 32
NPAIR = EW // (2 * CHUNK)
UNROLL = 13
NOUTER = NPAIR // UNROLL
TAIL = EW - NPAIR * 2 * CHUNK
RPT = 640
NPAD = RPT * NS

_MESH = plsc.VectorSubcoreMesh(core_axis_name="c", subcore_axis_name="s")


@functools.partial(
    pl.kernel,
    out_type=jax.ShapeDtypeStruct((NC, NPAD, D), jnp.float32),
    mesh=_MESH,
    scratch_types=[
        pltpu.VMEM_SHARED((NPAD, D), jnp.float32),
        [pltpu.VMEM((CHUNK,), jnp.int32) for _ in range(4)],
        pltpu.VMEM((TAIL,), jnp.int32),
        pltpu.VMEM((CHUNK, D), jnp.float32),
        pltpu.VMEM((CHUNK, D), jnp.float32),
        [pltpu.SemaphoreType.DMA for _ in range(4)],
    ],
)
def _sc_degree(edge_hbm, ones_hbm, zeros_hbm, deg_out, deg_acc, di, dit,
               ones_v, stage_v, ss):
    cid = lax.axis_index("c")
    sid = lax.axis_index("s")
    wid = sid * NC + cid
    base = pl.multiple_of(wid * EW, 16)
    pltpu.sync_copy(zeros_hbm, stage_v)
    pltpu.sync_copy(ones_hbm, ones_v)
    zh = [pltpu.async_copy(stage_v,
                           deg_acc.at[pl.ds(sid * RPT + k * CHUNK, CHUNK)],
                           ss[k % 4])
          for k in range(RPT // CHUNK)]
    for h in zh:
        h.wait()
    plsc.subcore_barrier()

    def body(i, carry):
        jb = pl.multiple_of(base + i * UNROLL * 2 * CHUNK, 16)
        hs = [None] * UNROLL
        for k in range(UNROLL):
            a, b = 2 * (k % 2), 2 * (k % 2) + 1
            if k >= 2:
                hs[k - 2][0].wait()
                hs[k - 2][1].wait()
            j = pl.multiple_of(jb + k * 2 * CHUNK, 16)
            pltpu.sync_copy(edge_hbm.at[pl.ds(E + j, CHUNK)], di[a])
            pltpu.sync_copy(edge_hbm.at[pl.ds(E + j + CHUNK, CHUNK)], di[b])
            cs0 = pltpu.async_copy(ones_v, deg_acc.at[di[a]], ss[a], add=True)
            cs1 = pltpu.async_copy(ones_v, deg_acc.at[di[b]], ss[b], add=True)
            hs[k] = (cs0, cs1)
        for k in (UNROLL - 2, UNROLL - 1):
            hs[k][0].wait()
            hs[k][1].wait()
        return carry

    lax.fori_loop(0, NOUTER, body, 0)
    pltpu.sync_copy(edge_hbm.at[pl.ds(E + base + 2 * NPAIR * CHUNK, TAIL)], dit)
    cst = pltpu.async_copy(ones_v.at[pl.ds(0, TAIL)], deg_acc.at[dit], ss[0],
                           add=True)
    cst.wait()
    plsc.subcore_barrier()

    obuf = [stage_v, ones_v]
    oh = [None] * (RPT // CHUNK)
    for k in range(RPT // CHUNK):
        if k >= 2:
            oh[k - 2].wait()
        r = pl.ds(sid * RPT + k * CHUNK, CHUNK)
        pltpu.sync_copy(deg_acc.at[r], obuf[k % 2])
        oh[k] = pltpu.async_copy(obuf[k % 2], deg_out.at[cid, r], ss[k % 2])
    for k in (RPT // CHUNK - 2, RPT // CHUNK - 1):
        oh[k].wait()


@functools.partial(
    pl.kernel,
    out_type=jax.ShapeDtypeStruct((NC, NPAD, D), jnp.float32),
    mesh=_MESH,
    scratch_types=[
        pltpu.VMEM_SHARED((NPAD, D), jnp.float32),
        [pltpu.VMEM((CHUNK,), jnp.int32) for _ in range(4)],
        [pltpu.VMEM((CHUNK,), jnp.int32) for _ in range(4)],
        pltpu.VMEM((TAIL,), jnp.int32),
        pltpu.VMEM((TAIL,), jnp.int32),
        [pltpu.VMEM((CHUNK, D), jnp.float32) for _ in range(4)],
        pltpu.VMEM((TAIL, D), jnp.float32),
        [pltpu.SemaphoreType.DMA for _ in range(4)],
        [pltpu.SemaphoreType.DMA for _ in range(4)],
    ],
)
def _sc_propagate(edge_hbm, xs_hbm, zeros_hbm, out_hbm,
                  acc, si, di, sit, dit, rb, rt, gs, ss):
    cid = lax.axis_index("c")
    sid = lax.axis_index("s")
    wid = sid * NC + cid
    base = pl.multiple_of(wid * EW, 16)
    pltpu.sync_copy(zeros_hbm, rb[0])
    pltpu.sync_copy(zeros_hbm, rb[1])
    zh = [pltpu.async_copy(rb[k % 2],
                           acc.at[pl.ds(sid * RPT + k * CHUNK, CHUNK)],
                           gs[k % 4])
          for k in range(RPT // CHUNK)]
    for h in zh:
        h.wait()
    plsc.subcore_barrier()

    def body(i, carry):
        jb = pl.multiple_of(base + i * UNROLL * 2 * CHUNK, 16)
        hs = [None] * UNROLL
        for k in range(UNROLL):
            a, b = 2 * (k % 2), 2 * (k % 2) + 1
            if k >= 2:
                hs[k - 2][0].wait()
                hs[k - 2][1].wait()
            j = pl.multiple_of(jb + k * 2 * CHUNK, 16)
            pltpu.sync_copy(edge_hbm.at[pl.ds(j, CHUNK)], si[a])
            pltpu.sync_copy(edge_hbm.at[pl.ds(j + CHUNK, CHUNK)], si[b])
            cg0 = pltpu.async_copy(xs_hbm.at[si[a]], rb[a], gs[a])
            cg1 = pltpu.async_copy(xs_hbm.at[si[b]], rb[b], gs[b])
            pltpu.sync_copy(edge_hbm.at[pl.ds(E + j, CHUNK)], di[a])
            pltpu.sync_copy(edge_hbm.at[pl.ds(E + j + CHUNK, CHUNK)], di[b])
            cg0.wait()
            cs0 = pltpu.async_copy(rb[a], acc.at[di[a]], ss[a], add=True)
            cg1.wait()
            cs1 = pltpu.async_copy(rb[b], acc.at[di[b]], ss[b], add=True)
            hs[k] = (cs0, cs1)
        for k in (UNROLL - 2, UNROLL - 1):
            hs[k][0].wait()
            hs[k][1].wait()
        return carry

    lax.fori_loop(0, NOUTER, body, 0)
    jt = pl.multiple_of(base + 2 * NPAIR * CHUNK, 16)
    pltpu.sync_copy(edge_hbm.at[pl.ds(jt, TAIL)], sit)
    cgt = pltpu.async_copy(xs_hbm.at[sit], rt, gs[0])
    pltpu.sync_copy(edge_hbm.at[pl.ds(E + jt, TAIL)], dit)
    cgt.wait()
    cst = pltpu.async_copy(rt, acc.at[dit], ss[0], add=True)
    cst.wait()
    plsc.subcore_barrier()

    oh = [None] * (RPT // CHUNK)
    for k in range(RPT // CHUNK):
        if k >= 4:
            oh[k - 4].wait()
        r = pl.ds(sid * RPT + k * CHUNK, CHUNK)
        pltpu.sync_copy(acc.at[r], rb[k % 4])
        oh[k] = pltpu.async_copy(rb[k % 4], out_hbm.at[cid, r], ss[k % 4])
    for k in range(RPT // CHUNK - 4, RPT // CHUNK):
        oh[k].wait()


_TR = 2000
_TG = N // _TR

_DOT = dict(preferred_element_type=jnp.float32, precision=lax.Precision.HIGHEST)


def _tc_mm_body(x_ref, w_ref, o_ref):
    o_ref[...] = jnp.dot(x_ref[...], w_ref[...], **_DOT)


def _tc_scale_body(dega_ref, degb_ref, xw_ref, dinv_ref, xs_ref):
    deg = dega_ref[0, :, :1] + degb_ref[0, :, :1] + 1.0
    dinv = lax.rsqrt(deg)
    dinv_ref[...] = jnp.broadcast_to(dinv, (_TR, D))
    xs_ref[...] = xw_ref[...] * dinv


def _tc_relu_mm2_body(p1a_ref, p1b_ref, xs1_ref, dinv_ref, w2_ref, b1_ref,
                      xs2_ref):
    dinv = dinv_ref[...]
    s = dinv * (p1a_ref[0] + p1b_ref[0] + xs1_ref[...]) + b1_ref[...]
    h = jnp.maximum(s, 0.0)
    xs2_ref[...] = jnp.dot(h, w2_ref[...], **_DOT) * dinv


def _tc_final_body(p2a_ref, p2b_ref, xs2_ref, dinv_ref, b2_ref, z_ref):
    z_ref[...] = (dinv_ref[...] * (p2a_ref[0] + p2b_ref[0] + xs2_ref[...])
                  + b2_ref[...])


def _row_spec(width):
    return pl.BlockSpec((_TR, width), lambda i: (i, 0))


def _part_spec(core):
    return pl.BlockSpec((1, _TR, D), lambda i, c=core: (c, i, 0))


def _full_spec(shape):
    return pl.BlockSpec(shape, lambda i: (0,) * len(shape))


_tc_mm = pl.pallas_call(
    _tc_mm_body,
    grid=(_TG,),
    in_specs=[_row_spec(D), _full_spec((D, D))],
    out_specs=_row_spec(D),
    out_shape=jax.ShapeDtypeStruct((N, D), jnp.float32),
)

_tc_scale = pl.pallas_call(
    _tc_scale_body,
    grid=(_TG,),
    in_specs=[_part_spec(0), _part_spec(1), _row_spec(D)],
    out_specs=[_row_spec(D), _row_spec(D)],
    out_shape=[jax.ShapeDtypeStruct((N, D), jnp.float32)] * 2,
)

_tc_relu_mm2 = pl.pallas_call(
    _tc_relu_mm2_body,
    grid=(_TG,),
    in_specs=[_part_spec(0), _part_spec(1), _row_spec(D), _row_spec(D),
              _full_spec((D, D)), _full_spec((1, D))],
    out_specs=_row_spec(D),
    out_shape=jax.ShapeDtypeStruct((N, D), jnp.float32),
)

_tc_final = pl.pallas_call(
    _tc_final_body,
    grid=(_TG,),
    in_specs=[_part_spec(0), _part_spec(1), _row_spec(D), _row_spec(D),
              _full_spec((1, D))],
    out_specs=_row_spec(D),
    out_shape=jax.ShapeDtypeStruct((N, D), jnp.float32),
)


def kernel(x, edge_index, W1, b1, W2, b2):
    edges = edge_index.astype(jnp.int32).reshape(-1)

    onesd = jnp.ones((CHUNK, D), jnp.float32)
    zerosd = jnp.zeros((CHUNK, D), jnp.float32)

    xw1 = _tc_mm(x, W1)
    degp = _sc_degree(edges, onesd, zerosd)
    dinv128, xs1 = _tc_scale(degp, degp, xw1)
    p1 = _sc_propagate(edges, xs1, zerosd)
    xs2 = _tc_relu_mm2(p1, p1, xs1, dinv128, W2, b1.reshape(1, D))
    p2 = _sc_propagate(edges, xs2, zerosd)
    return _tc_final(p2, p2, xs2, dinv128, b2.reshape(1, D))


# speedup vs baseline: 1.5110x; 1.5110x over previous
import functools

import jax
import jax.numpy as jnp
from jax import lax
from jax.experimental import pallas as pl
from jax.experimental.pallas import tpu as pltpu
from jax.experimental.pallas import tpu_sc as plsc

N = 10000
E = 320000
D = 128
NC = 2
NS = 16
NW = NC * NS
EW = E // NW
CHUNK = 64
NPAIR = EW // (2 * CHUNK)
UNROLL = 13
NOUTER = NPAIR // UNROLL
TAIL = EW - NPAIR * 2 * CHUNK
RPT = 640
NPAD = RPT * NS

_MESH = plsc.VectorSubcoreMesh(core_axis_name="c", subcore_axis_name="s")


@functools.partial(
    pl.kernel,
    out_type=jax.ShapeDtypeStruct((NC, NPAD, D), jnp.float32),
    mesh=_MESH,
    scratch_types=[
        pltpu.VMEM_SHARED((NPAD, D), jnp.float32),
        [pltpu.VMEM((CHUNK,), jnp.int32) for _ in range(4)],
        pltpu.VMEM((TAIL,), jnp.int32),
        pltpu.VMEM((CHUNK, D), jnp.float32),
        pltpu.VMEM((CHUNK, D), jnp.float32),
        [pltpu.SemaphoreType.DMA for _ in range(4)],
    ],
)
def _sc_degree(edge_hbm, ones_hbm, zeros_hbm, deg_out, deg_acc, di, dit,
               ones_v, stage_v, ss):
    cid = lax.axis_index("c")
    sid = lax.axis_index("s")
    wid = sid * NC + cid
    base = pl.multiple_of(wid * EW, 16)
    pltpu.sync_copy(zeros_hbm, stage_v)
    pltpu.sync_copy(ones_hbm, ones_v)
    zh = [pltpu.async_copy(stage_v,
                           deg_acc.at[pl.ds(sid * RPT + k * CHUNK, CHUNK)],
                           ss[k % 4])
          for k in range(RPT // CHUNK)]
    for h in zh:
        h.wait()
    plsc.subcore_barrier()

    def body(i, carry):
        jb = pl.multiple_of(base + i * UNROLL * 2 * CHUNK, 16)
        hs = [None] * UNROLL
        for k in range(UNROLL):
            a, b = 2 * (k % 2), 2 * (k % 2) + 1
            if k >= 2:
                hs[k - 2][0].wait()
                hs[k - 2][1].wait()
            j = pl.multiple_of(jb + k * 2 * CHUNK, 16)
            pltpu.sync_copy(edge_hbm.at[pl.ds(E + j, CHUNK)], di[a])
            pltpu.sync_copy(edge_hbm.at[pl.ds(E + j + CHUNK, CHUNK)], di[b])
            cs0 = pltpu.async_copy(ones_v, deg_acc.at[di[a]], ss[a], add=True)
            cs1 = pltpu.async_copy(ones_v, deg_acc.at[di[b]], ss[b], add=True)
            hs[k] = (cs0, cs1)
        for k in (UNROLL - 2, UNROLL - 1):
            hs[k][0].wait()
            hs[k][1].wait()
        return carry

    lax.fori_loop(0, NOUTER, body, 0)
    pltpu.sync_copy(edge_hbm.at[pl.ds(E + base + 2 * NPAIR * CHUNK, TAIL)], dit)
    cst = pltpu.async_copy(ones_v.at[pl.ds(0, TAIL)], deg_acc.at[dit], ss[0],
                           add=True)
    cst.wait()
    plsc.subcore_barrier()

    obuf = [stage_v, ones_v]
    oh = [None] * (RPT // CHUNK)
    for k in range(RPT // CHUNK):
        if k >= 2:
            oh[k - 2].wait()
        r = pl.ds(sid * RPT + k * CHUNK, CHUNK)
        pltpu.sync_copy(deg_acc.at[r], obuf[k % 2])
        oh[k] = pltpu.async_copy(obuf[k % 2], deg_out.at[cid, r], ss[k % 2])
    for k in (RPT // CHUNK - 2, RPT // CHUNK - 1):
        oh[k].wait()


@functools.partial(
    pl.kernel,
    out_type=jax.ShapeDtypeStruct((NC, NPAD, D), jnp.float32),
    mesh=_MESH,
    scratch_types=[
        pltpu.VMEM_SHARED((NPAD, D), jnp.float32),
        [pltpu.VMEM((CHUNK,), jnp.int32) for _ in range(4)],
        [pltpu.VMEM((CHUNK,), jnp.int32) for _ in range(4)],
        pltpu.VMEM((TAIL,), jnp.int32),
        pltpu.VMEM((TAIL,), jnp.int32),
        [pltpu.VMEM((CHUNK, D), jnp.float32) for _ in range(4)],
        pltpu.VMEM((TAIL, D), jnp.float32),
        [pltpu.SemaphoreType.DMA for _ in range(4)],
        [pltpu.SemaphoreType.DMA for _ in range(4)],
    ],
)
def _sc_propagate(edge_hbm, xs_hbm, zeros_hbm, out_hbm,
                  acc, si, di, sit, dit, rb, rt, gs, ss):
    cid = lax.axis_index("c")
    sid = lax.axis_index("s")
    wid = sid * NC + cid
    base = pl.multiple_of(wid * EW, 16)
    pltpu.sync_copy(zeros_hbm, rb[0])
    pltpu.sync_copy(zeros_hbm, rb[1])
    zh = [pltpu.async_copy(rb[k % 2],
                           acc.at[pl.ds(sid * RPT + k * CHUNK, CHUNK)],
                           gs[k % 4])
          for k in range(RPT // CHUNK)]
    for h in zh:
        h.wait()
    plsc.subcore_barrier()

    def body(i, carry):
        jb = pl.multiple_of(base + i * UNROLL * 2 * CHUNK, 16)
        hs = [None] * UNROLL
        for k in range(UNROLL):
            a, b = 2 * (k % 2), 2 * (k % 2) + 1
            if k >= 2:
                hs[k - 2][0].wait()
                hs[k - 2][1].wait()
            j = pl.multiple_of(jb + k * 2 * CHUNK, 16)
            pltpu.sync_copy(edge_hbm.at[pl.ds(j, CHUNK)], si[a])
            pltpu.sync_copy(edge_hbm.at[pl.ds(j + CHUNK, CHUNK)], si[b])
            cg0 = pltpu.async_copy(xs_hbm.at[si[a]], rb[a], gs[a])
            cg1 = pltpu.async_copy(xs_hbm.at[si[b]], rb[b], gs[b])
            pltpu.sync_copy(edge_hbm.at[pl.ds(E + j, CHUNK)], di[a])
            pltpu.sync_copy(edge_hbm.at[pl.ds(E + j + CHUNK, CHUNK)], di[b])
            cg0.wait()
            cs0 = pltpu.async_copy(rb[a], acc.at[di[a]], ss[a], add=True)
            cg1.wait()
            cs1 = pltpu.async_copy(rb[b], acc.at[di[b]], ss[b], add=True)
            hs[k] = (cs0, cs1)
        for k in (UNROLL - 2, UNROLL - 1):
            hs[k][0].wait()
            hs[k][1].wait()
        return carry

    lax.fori_loop(0, NOUTER, body, 0)
    jt = pl.multiple_of(base + 2 * NPAIR * CHUNK, 16)
    pltpu.sync_copy(edge_hbm.at[pl.ds(jt, TAIL)], sit)
    cgt = pltpu.async_copy(xs_hbm.at[sit], rt, gs[0])
    pltpu.sync_copy(edge_hbm.at[pl.ds(E + jt, TAIL)], dit)
    cgt.wait()
    cst = pltpu.async_copy(rt, acc.at[dit], ss[0], add=True)
    cst.wait()
    plsc.subcore_barrier()

    oh = [None] * (RPT // CHUNK)
    for k in range(RPT // CHUNK):
        if k >= 4:
            oh[k - 4].wait()
        r = pl.ds(sid * RPT + k * CHUNK, CHUNK)
        pltpu.sync_copy(acc.at[r], rb[k % 4])
        oh[k] = pltpu.async_copy(rb[k % 4], out_hbm.at[cid, r], ss[k % 4])
    for k in range(RPT // CHUNK - 4, RPT // CHUNK):
        oh[k].wait()


_TR = 2000
_TG = N // _TR

_DOT = dict(preferred_element_type=jnp.float32, precision=lax.Precision.HIGHEST)


def _tc_mm_body(x_ref, w_ref, o_ref):
    o_ref[...] = jnp.dot(x_ref[...], w_ref[...], **_DOT)


def _tc_scale_body(dega_ref, degb_ref, xw_ref, dinv_ref, xs_ref):
    deg = dega_ref[0, :, :1] + degb_ref[0, :, :1] + 1.0
    dinv = lax.rsqrt(deg)
    dinv_ref[...] = jnp.broadcast_to(dinv, (_TR, D))
    xs_ref[...] = xw_ref[...] * dinv


def _tc_relu_mm2_body(p1a_ref, p1b_ref, xs1_ref, dinv_ref, w2_ref, b1_ref,
                      xs2_ref):
    dinv = dinv_ref[...]
    s = dinv * (p1a_ref[0] + p1b_ref[0] + xs1_ref[...]) + b1_ref[...]
    h = jnp.maximum(s, 0.0)
    xs2_ref[...] = jnp.dot(h, w2_ref[...], **_DOT) * dinv


def _tc_final_body(p2a_ref, p2b_ref, xs2_ref, dinv_ref, b2_ref, z_ref):
    z_ref[...] = (dinv_ref[...] * (p2a_ref[0] + p2b_ref[0] + xs2_ref[...])
                  + b2_ref[...])


def _row_spec(width):
    return pl.BlockSpec((_TR, width), lambda i: (i, 0))


def _part_spec(core):
    return pl.BlockSpec((1, _TR, D), lambda i, c=core: (c, i, 0))


def _full_spec(shape):
    return pl.BlockSpec(shape, lambda i: (0,) * len(shape))


_tc_mm = pl.pallas_call(
    _tc_mm_body,
    grid=(_TG,),
    in_specs=[_row_spec(D), _full_spec((D, D))],
    out_specs=_row_spec(D),
    out_shape=jax.ShapeDtypeStruct((N, D), jnp.float32),
)

_tc_scale = pl.pallas_call(
    _tc_scale_body,
    grid=(_TG,),
    in_specs=[_part_spec(0), _part_spec(1), _row_spec(D)],
    out_specs=[_row_spec(D), _row_spec(D)],
    out_shape=[jax.ShapeDtypeStruct((N, D), jnp.float32)] * 2,
)

_tc_relu_mm2 = pl.pallas_call(
    _tc_relu_mm2_body,
    grid=(_TG,),
    in_specs=[_part_spec(0), _part_spec(1), _row_spec(D), _row_spec(D),
              _full_spec((D, D)), _full_spec((1, D))],
    out_specs=_row_spec(D),
    out_shape=jax.ShapeDtypeStruct((N, D), jnp.float32),
)

_tc_final = pl.pallas_call(
    _tc_final_body,
    grid=(_TG,),
    in_specs=[_part_spec(0), _part_spec(1), _row_spec(D), _row_spec(D),
              _full_spec((1, D))],
    out_specs=_row_spec(D),
    out_shape=jax.ShapeDtypeStruct((N, D), jnp.float32),
)


def kernel(x, edge_index, W1, b1, W2, b2):
    edges = edge_index.astype(jnp.int32).reshape(-1)

    onesd = jnp.ones((CHUNK, D), jnp.float32)
    zerosd = jnp.zeros((CHUNK, D), jnp.float32)

    xw1 = _tc_mm(x, W1)
    degp = _sc_degree(edges, onesd, zerosd)
    dinv128, xs1 = _tc_scale(degp, degp, xw1)
    p1 = _sc_propagate(edges, xs1, zerosd)
    xs2 = _tc_relu_mm2(p1, p1, xs1, dinv128, W2, b1.reshape(1, D))
    p2 = _sc_propagate(edges, xs2, zerosd)
    return _tc_final(p2, p2, xs2, dinv128, b2.reshape(1, D))
